# baseline (speedup 1.0000x reference)
import functools

import jax
import jax.numpy as jnp
from jax import lax
from jax.experimental import pallas as pl
from jax.experimental.pallas import tpu as pltpu
from jax.experimental.pallas import tpu_sc as plsc


def _make_emb(N, V, D, NC, NS):
    NW = NC * NS
    n_per_w = N // NW
    CH = 32
    NBUF = 4
    n_chunks = n_per_w // CH
    n_outer = n_chunks // NBUF
    mesh = plsc.VectorSubcoreMesh(core_axis_name="c", subcore_axis_name="s")

    @functools.partial(
        pl.kernel,
        mesh=mesh,
        out_type=jax.ShapeDtypeStruct((N, D), jnp.float32),
        scratch_types=[
            pltpu.VMEM((n_per_w,), jnp.int32),
            pltpu.VMEM((NBUF, CH, D), jnp.float32),
        ] + [pltpu.SemaphoreType.DMA] * (2 * NBUF),
    )
    def emb(tok_hbm, table_hbm, out_hbm, idx_all, rows_v, *sems):
        gsems, ssems = sems[:NBUF], sems[NBUF:]
        wid = lax.axis_index("s") * NC + lax.axis_index("c")
        base = wid * n_per_w
        pltpu.sync_copy(tok_hbm.at[pl.ds(base, n_per_w)], idx_all)

        def g_start(i, b):
            pltpu.async_copy(
                table_hbm.at[idx_all.at[pl.ds(i * CH, CH)]],
                rows_v.at[b], gsems[b])

        def g_wait(i, b):
            pltpu.make_async_copy(
                table_hbm.at[idx_all.at[pl.ds(i * CH, CH)]],
                rows_v.at[b], gsems[b]).wait()

        def s_start(i, b):
            pltpu.async_copy(
                rows_v.at[b], out_hbm.at[pl.ds(base + i * CH, CH)], ssems[b])

        def s_wait(i, b):
            pltpu.make_async_copy(
                rows_v.at[b], out_hbm.at[pl.ds(base + i * CH, CH)],
                ssems[b]).wait()

        for b in range(NBUF):
            g_start(b, b)

        def outer(g2, _):
            i0 = g2 * NBUF
            for b in range(NBUF):
                g_wait(i0 + b, b)
                s_start(i0 + b, b)
            for b in range(NBUF):
                s_wait(i0 + b, b)

                @pl.when(g2 < n_outer - 1)
                def _():
                    g_start(i0 + NBUF + b, b)

            return 0

        lax.fori_loop(0, n_outer, outer, 0)

    return emb


def kernel(tokens, W_E):
    B, S = tokens.shape
    V, D = W_E.shape
    N = B * S
    info = plsc.get_sparse_core_info()
    emb = _make_emb(N, V, D, info.num_cores, info.num_subcores)
    out = emb(tokens.reshape(N).astype(jnp.int32), W_E)
    return out.reshape(B, S, D)


# speedup vs baseline: 1.0418x; 1.0418x over previous
import functools

import jax
import jax.numpy as jnp
from jax import lax
from jax.experimental import pallas as pl
from jax.experimental.pallas import tpu as pltpu
from jax.experimental.pallas import tpu_sc as plsc


def _make_emb(B, S, V, D, NC, NS):
    NW = NC * NS
    N = B * S
    n_per_w = N // NW
    w_per_row = S // n_per_w
    CH = 32
    NBUF = 4
    n_chunks = n_per_w // CH
    n_outer = n_chunks // NBUF
    mesh = plsc.VectorSubcoreMesh(core_axis_name="c", subcore_axis_name="s")

    @functools.partial(
        pl.kernel,
        mesh=mesh,
        out_type=jax.ShapeDtypeStruct((B, S, D), jnp.float32),
        scratch_types=[
            pltpu.VMEM((n_per_w,), jnp.int32),
            pltpu.VMEM((NBUF, CH, D), jnp.float32),
        ] + [pltpu.SemaphoreType.DMA] * (2 * NBUF),
    )
    def emb(tok_hbm, table_hbm, out_hbm, idx_all, rows_v, *sems):
        gsems, ssems = sems[:NBUF], sems[NBUF:]
        wid = lax.axis_index("s") * NC + lax.axis_index("c")
        bi = wid // w_per_row
        off = (wid % w_per_row) * n_per_w
        pltpu.sync_copy(tok_hbm.at[bi, pl.ds(off, n_per_w)], idx_all)

        def g_start(i, b):
            pltpu.async_copy(
                table_hbm.at[idx_all.at[pl.ds(i * CH, CH)]],
                rows_v.at[b], gsems[b])

        def g_wait(i, b):
            pltpu.make_async_copy(
                table_hbm.at[idx_all.at[pl.ds(i * CH, CH)]],
                rows_v.at[b], gsems[b]).wait()

        def s_start(i, b):
            pltpu.async_copy(
                rows_v.at[b], out_hbm.at[bi, pl.ds(off + i * CH, CH)],
                ssems[b])

        def s_wait(i, b):
            pltpu.make_async_copy(
                rows_v.at[b], out_hbm.at[bi, pl.ds(off + i * CH, CH)],
                ssems[b]).wait()

        g_start(0, 0)
        g_start(1, 1)

        def outer(g2, _):
            i0 = g2 * NBUF
            for b in range(NBUF):
                i = i0 + b
                g_wait(i, b)
                s_start(i, b)
                if b < 2:
                    @pl.when(g2 == 0)
                    def _():
                        g_start(i + 2, (b + 2) % NBUF)

                    @pl.when(g2 > 0)
                    def _():
                        s_wait(i - 2, (b - 2) % NBUF)
                        g_start(i + 2, (b + 2) % NBUF)
                else:
                    s_wait(i - 2, (b - 2) % NBUF)

                    @pl.when(g2 < n_outer - 1)
                    def _():
                        g_start(i + 2, (b + 2) % NBUF)

            return 0

        lax.fori_loop(0, n_outer, outer, 0)
        s_wait(n_chunks - 2, (n_chunks - 2) % NBUF)
        s_wait(n_chunks - 1, (n_chunks - 1) % NBUF)

    return emb


def kernel(tokens, W_E):
    B, S = tokens.shape
    V, D = W_E.shape
    info = plsc.get_sparse_core_info()
    emb = _make_emb(B, S, V, D, info.num_cores, info.num_subcores)
    return emb(tokens, W_E)


# speedup vs baseline: 1.0443x; 1.0024x over previous
import functools

import jax
import jax.numpy as jnp
from jax import lax
from jax.experimental import pallas as pl
from jax.experimental.pallas import tpu as pltpu
from jax.experimental.pallas import tpu_sc as plsc


def _make_emb(B, S, V, D, NC, NS):
    NW = NC * NS
    N = B * S
    n_per_w = N // NW
    w_per_row = S // n_per_w
    CH = 64
    NBUF = 2
    n_chunks = n_per_w // CH
    n_outer = n_chunks // NBUF
    mesh = plsc.VectorSubcoreMesh(core_axis_name="c", subcore_axis_name="s")

    @functools.partial(
        pl.kernel,
        mesh=mesh,
        out_type=jax.ShapeDtypeStruct((B, S, D), jnp.float32),
        scratch_types=[
            pltpu.VMEM((n_per_w,), jnp.int32),
            pltpu.VMEM((NBUF, CH, D), jnp.float32),
        ] + [pltpu.SemaphoreType.DMA] * (2 * NBUF),
    )
    def emb(tok_hbm, table_hbm, out_hbm, idx_all, rows_v, *sems):
        gsems, ssems = sems[:NBUF], sems[NBUF:]
        wid = lax.axis_index("s") * NC + lax.axis_index("c")
        bi = wid // w_per_row
        off = (wid % w_per_row) * n_per_w
        pltpu.sync_copy(tok_hbm.at[bi, pl.ds(off, n_per_w)], idx_all)

        def g_start(i, b):
            pltpu.async_copy(
                table_hbm.at[idx_all.at[pl.ds(i * CH, CH)]],
                rows_v.at[b], gsems[b])

        def g_wait(i, b):
            pltpu.make_async_copy(
                table_hbm.at[idx_all.at[pl.ds(i * CH, CH)]],
                rows_v.at[b], gsems[b]).wait()

        def s_start(i, b):
            pltpu.async_copy(
                rows_v.at[b], out_hbm.at[bi, pl.ds(off + i * CH, CH)],
                ssems[b])

        def s_wait(i, b):
            pltpu.make_async_copy(
                rows_v.at[b], out_hbm.at[bi, pl.ds(off + i * CH, CH)],
                ssems[b]).wait()

        g_start(0, 0)
        g_start(1, 1)

        def outer(g2, _):
            i0 = g2 * NBUF
            for b in range(NBUF):
                i = i0 + b
                g_wait(i, b)
                s_start(i, b)
                @pl.when(g2 < n_outer - 1)
                def _():
                    s_wait(i, b)
                    g_start(i + NBUF, b)

            return 0

        lax.fori_loop(0, n_outer, outer, 0)
        s_wait(n_chunks - 2, 0)
        s_wait(n_chunks - 1, 1)

    return emb


def kernel(tokens, W_E):
    B, S = tokens.shape
    V, D = W_E.shape
    info = plsc.get_sparse_core_info()
    emb = _make_emb(B, S, V, D, info.num_cores, info.num_subcores)
    return emb(tokens, W_E)
